# Initial kernel scaffold; baseline (speedup 1.0000x reference)
import jax
import jax.numpy as jnp
from jax.experimental import pallas as pl


def kernel(input, emb, W, b):
    raise NotImplementedError("write your pallas kernel here")


# speedup vs baseline: 6.1239x; 6.1239x over previous
import jax
import jax.numpy as jnp
from jax import lax
from jax.experimental import pallas as pl
from jax.experimental.pallas import tpu as pltpu
from jax.experimental.pallas import tpu_sc as plsc

V = 100000
D = 200
S = 100
B = 4096
C = 2

NC = 2
NS = 16
NW = NC * NS
BW = B // NW
L = 16


def _mm_body(emb_ref, wr_ref, t_ref):
    t_ref[...] = jnp.dot(emb_ref[...], wr_ref[...],
                         preferred_element_type=jnp.float32
                         ).astype(jnp.bfloat16)


_MM_BLOCK = 2000


def _tc_table(emb, wr):
    return pl.pallas_call(
        _mm_body,
        grid=(V // _MM_BLOCK,),
        in_specs=[
            pl.BlockSpec((_MM_BLOCK, D), lambda i: (i, 0)),
            pl.BlockSpec((D, S * C), lambda i: (0, 0)),
        ],
        out_specs=pl.BlockSpec((_MM_BLOCK, S * C), lambda i: (i, 0)),
        out_shape=jax.ShapeDtypeStruct((V, S * C), jnp.bfloat16),
    )(emb, wr)


_K = 10


def _sc_body(t_hbm, idx_hbm, out_hbm, idx_v, rows_v, out_v, sem):
    wid = lax.axis_index("s") * NC + lax.axis_index("c")
    base = wid * BW
    pltpu.sync_copy(idx_hbm.at[:, pl.ds(base, BW)], idx_v)

    def chunk(k, _):
        s0 = k * _K
        for u in range(_K):
            pltpu.async_copy(t_hbm.at[idx_v.at[s0 + u]],
                             rows_v.at[pl.ds((s0 + u) * BW, BW)], sem)
        for u in range(_K):
            pltpu.make_async_copy(t_hbm.at[idx_v.at[s0 + u]],
                                  rows_v.at[pl.ds((s0 + u) * BW, BW)],
                                  sem).wait()
        return 0

    lax.fori_loop(0, S // _K, chunk, 0)

    nq = BW // L
    himask = jnp.full((L,), -65536, dtype=jnp.int32)

    def body(s, accs):
        a0, a1 = accs
        new0, new1 = [], []
        for q in range(nq):
            w = rows_v[pl.ds(s * BW + q * L, L)]
            lo = lax.bitcast_convert_type(w << 16, jnp.float32)
            hi = lax.bitcast_convert_type(w & himask, jnp.float32)
            new0.append(a0[q] + lo)
            new1.append(a1[q] + hi)
        return tuple(new0), tuple(new1)

    zeros = tuple(jnp.zeros((L,), jnp.float32) for _ in range(nq))
    acc0, acc1 = lax.fori_loop(0, S, body, (zeros, zeros))
    for q in range(nq):
        out_v[pl.ds(q * L, L)] = acc0[q]
        out_v[pl.ds(BW + q * L, L)] = acc1[q]
    pltpu.sync_copy(out_v.at[pl.ds(0, BW)], out_hbm.at[pl.ds(base, BW)])
    pltpu.sync_copy(out_v.at[pl.ds(BW, BW)], out_hbm.at[pl.ds(B + base, BW)])


def _sc_gather_reduce(t_flat, idx):
    k = pl.kernel(
        _sc_body,
        out_type=jax.ShapeDtypeStruct((C * B,), jnp.float32),
        mesh=plsc.VectorSubcoreMesh(core_axis_name="c", subcore_axis_name="s"),
        scratch_types=[
            pltpu.VMEM((S, BW), jnp.int32),
            pltpu.VMEM((S * BW,), jnp.int32),
            pltpu.VMEM((C * BW,), jnp.float32),
            pltpu.SemaphoreType.DMA,
        ],
    )
    return k(t_flat, idx)


def _lsm_body(x_ref, b_ref, o_ref):
    y = x_ref[...] + b_ref[...]
    m = jnp.max(y, axis=-1, keepdims=True)
    z = y - m
    o_ref[...] = z - jnp.log(jnp.sum(jnp.exp(z), axis=-1, keepdims=True))


def _tc_logsoftmax(logits, b2):
    return pl.pallas_call(
        _lsm_body,
        in_specs=[
            pl.BlockSpec((B, C), lambda: (0, 0)),
            pl.BlockSpec((1, C), lambda: (0, 0)),
        ],
        out_specs=pl.BlockSpec((B, C), lambda: (0, 0)),
        out_shape=jax.ShapeDtypeStruct((B, C), jnp.float32),
    )(logits, b2)


def kernel(input, emb, W, b):
    wr = W.reshape(S, D, C).transpose(1, 0, 2).reshape(D, S * C)
    idx = (input * S + lax.broadcasted_iota(jnp.int32, (1, S), 1)).T
    t = _tc_table(emb, wr)
    t_packed = lax.bitcast_convert_type(
        t.reshape(V, S, C), jnp.int32).reshape(V * S)
    logits = _sc_gather_reduce(t_packed, idx)
    logits = logits.reshape(C, B).T
    return _tc_logsoftmax(logits, b.reshape(1, C))
